# Initial kernel scaffold; baseline (speedup 1.0000x reference)
import jax
import jax.numpy as jnp
from jax.experimental import pallas as pl


def kernel(x, edge_index, batch, params):
    raise NotImplementedError("write your pallas kernel here")


# speedup vs baseline: 3.9471x; 3.9471x over previous
import jax
import jax.numpy as jnp
from jax import lax
from jax.experimental import pallas as pl
from jax.experimental.pallas import tpu as pltpu
from jax.experimental.pallas import tpu_sc as plsc

N = 10000
E = 320000
DF = 128
DE = 64
G = 64
DT = 10
NUM_LAYERS = 4
BN_EPS = 1e-5

NC = 2
NS = 16
NW = NC * NS

CH = 128
EPW = ((E // NW + CH - 1) // CH) * CH
EP = EPW * NW
NP = ((N + 127) // 128) * 128
APT = NP // NS
RPT = 320
NP2 = RPT * NW
SEG = 72

_f32 = jnp.float32


def _make_scatter(D):
    def body(h_hbm, src_hbm, dst_hbm, zero_hbm, out_hbm,
             idx_s, idx_d, rows, acc, sem):
        c = lax.axis_index("c")
        s = lax.axis_index("s")
        wid = c * NS + s
        pltpu.sync_copy(zero_hbm.at[pl.ds(s * APT, APT)],
                        acc.at[pl.ds(s * APT, APT)])
        plsc.subcore_barrier()
        base = wid * EPW

        def chunk(k, carry):
            off = base + k * CH
            pltpu.sync_copy(src_hbm.at[pl.ds(off, CH)], idx_s)
            pltpu.sync_copy(dst_hbm.at[pl.ds(off, CH)], idx_d)
            pltpu.async_copy(h_hbm.at[idx_s], rows, sem).wait()
            pltpu.sync_copy(rows, acc.at[idx_d], add=True)
            return carry

        lax.fori_loop(0, EPW // CH, chunk, 0)
        plsc.subcore_barrier()
        pltpu.sync_copy(acc.at[pl.ds(s * APT, APT)],
                        out_hbm.at[c, pl.ds(s * APT, APT)])

    return pl.kernel(
        body,
        out_type=jax.ShapeDtypeStruct((NC, NP, D), _f32),
        mesh=plsc.VectorSubcoreMesh(core_axis_name="c", subcore_axis_name="s"),
        scratch_types=[
            pltpu.VMEM((CH,), jnp.int32),
            pltpu.VMEM((CH,), jnp.int32),
            pltpu.VMEM((CH, D), _f32),
            pltpu.VMEM_SHARED((NP, D), _f32),
            pltpu.SemaphoreType.DMA,
        ],
        compiler_params=pltpu.CompilerParams(use_tc_tiling_on_sc=False),
    )


_sc_scatter128 = _make_scatter(DF)
_sc_scatter64 = _make_scatter(DE)


def _sc_segmax_body(h0, h1, h2, h3, bat_hbm, o0, o1, o2, o3, batv, buf, acc):
    c = lax.axis_index("c")
    s = lax.axis_index("s")
    wid = c * NS + s
    r0 = wid * RPT
    pltpu.sync_copy(bat_hbm.at[pl.ds(r0, RPT)], batv.at[pl.ds(0, RPT)])
    neg = jnp.full((16,), -jnp.inf, dtype=_f32)
    for h_hbm, o_hbm in ((h0, o0), (h1, o1), (h2, o2), (h3, o3)):
        pltpu.sync_copy(h_hbm.at[pl.ds(r0, RPT)], buf)

        def init(i, carry):
            for q in range(DE // 16):
                acc[i, pl.ds(q * 16, 16)] = neg
            return carry

        lax.fori_loop(0, SEG, init, 0)

        def row(j, carry):
            g = batv[pl.ds(j, 16)][0]
            for q in range(DE // 16):
                acc[g, pl.ds(q * 16, 16)] = jnp.maximum(
                    acc[g, pl.ds(q * 16, 16)], buf[j, pl.ds(q * 16, 16)])
            return carry

        lax.fori_loop(0, RPT, row, 0)
        pltpu.sync_copy(acc, o_hbm.at[wid])


_pool_shape = jax.ShapeDtypeStruct((NW, SEG, DE), _f32)
_sc_segmax = pl.kernel(
    _sc_segmax_body,
    out_type=[_pool_shape, _pool_shape, _pool_shape, _pool_shape],
    mesh=plsc.VectorSubcoreMesh(core_axis_name="c", subcore_axis_name="s"),
    scratch_types=[
        pltpu.VMEM((RPT + 16,), jnp.int32),
        pltpu.VMEM((RPT, DE), _f32),
        pltpu.VMEM((SEG, DE), _f32),
    ],
    compiler_params=pltpu.CompilerParams(use_tc_tiling_on_sc=False),
)


def _dot(a, b):
    return jnp.dot(a.astype(jnp.bfloat16), b.astype(jnp.bfloat16),
                   preferred_element_type=_f32)


def _bn_relu(a, g, b):
    mu = jnp.mean(a, axis=0, keepdims=True)
    d = a - mu
    var = jnp.mean(d * d, axis=0, keepdims=True)
    return jnp.maximum(g * (d * lax.rsqrt(var + BN_EPS)) + b, 0.0)


def _tc_layer_body(h_ref, p_ref, w1_ref, b1_ref, g1_ref, be1_ref, w2_ref,
                   b2_ref, g2_ref, be2_ref, fcw_ref, fcb_ref, h_out_ref):
    a = h_ref[pl.ds(0, N), :] + p_ref[0, :N, :] + p_ref[1, :N, :]
    a = _dot(a, w1_ref[...]) + b1_ref[...]
    a = _bn_relu(a, g1_ref[...], be1_ref[...])
    a = _dot(a, w2_ref[...]) + b2_ref[...]
    a = _bn_relu(a, g2_ref[...], be2_ref[...])
    h = jnp.maximum(_dot(a, fcw_ref[...]) + fcb_ref[...], 0.0)
    h_out_ref[pl.ds(0, N), :] = h
    h_out_ref[pl.ds(N, NP2 - N), :] = jnp.zeros((NP2 - N, DE), _f32)


def _tc_layer(h, p, cp, fcw, fcb):
    return pl.pallas_call(
        _tc_layer_body,
        out_shape=jax.ShapeDtypeStruct((NP2, DE), _f32),
    )(h, p, cp['W1'], cp['b1'].reshape(1, DE), cp['g1'].reshape(1, DE),
      cp['be1'].reshape(1, DE), cp['W2'], cp['b2'].reshape(1, DE),
      cp['g2'].reshape(1, DE), cp['be2'].reshape(1, DE),
      fcw, fcb.reshape(1, DE))


def _tc_head_body(p0_ref, p1_ref, p2_ref, p3_ref, w1_ref, b1_ref,
                  w2_ref, b2_ref, o_ref):
    pools = [jnp.max(r[...][:, :G, :], axis=0)
             for r in (p0_ref, p1_ref, p2_ref, p3_ref)]
    hc = jnp.concatenate(pools, axis=1)
    t = jnp.maximum(_dot(hc, w1_ref[...]) + b1_ref[...], 0.0)
    o_ref[...] = _dot(t, w2_ref[...]) + b2_ref[...]


def _tc_head(pools, params):
    return pl.pallas_call(
        _tc_head_body,
        out_shape=jax.ShapeDtypeStruct((G, DT), _f32),
    )(*pools, params['fc1_W'], params['fc1_b'].reshape(1, DE),
      params['fc2_W'], params['fc2_b'].reshape(1, DT))


def kernel(x, edge_index, batch, params):
    src = edge_index[0].astype(jnp.int32)
    dst = edge_index[1].astype(jnp.int32)
    srcp = jnp.concatenate([src, jnp.zeros((EP - E,), jnp.int32)])
    dstp = jnp.concatenate([dst, jnp.full((EP - E,), N, jnp.int32)])
    batp = jnp.concatenate([batch.astype(jnp.int32),
                            jnp.full((NP2 - N,), G, jnp.int32)])
    zeros128 = jnp.zeros((NP, DF), _f32)
    zeros64 = jnp.zeros((NP, DE), _f32)

    convs = params['convs']
    hs = []
    h = x
    for i in range(NUM_LAYERS):
        if i == 0:
            p = _sc_scatter128(h, srcp, dstp, zeros128)
        else:
            p = _sc_scatter64(h, srcp, dstp, zeros64)
        h = _tc_layer(h, p, convs[i], params['fc_max_W'], params['fc_max_b'])
        hs.append(h)

    pools = _sc_segmax(hs[0], hs[1], hs[2], hs[3], batp)
    return _tc_head(pools, params)


# speedup vs baseline: 6.0571x; 1.5346x over previous
import jax
import jax.numpy as jnp
from jax import lax
from jax.experimental import pallas as pl
from jax.experimental.pallas import tpu as pltpu
from jax.experimental.pallas import tpu_sc as plsc

N = 10000
E = 320000
DF = 128
DE = 64
G = 64
DT = 10
NUM_LAYERS = 4
BN_EPS = 1e-5

NC = 2
NS = 16
NW = NC * NS

CH = 128
EPW = ((E // NW + CH - 1) // CH) * CH
EP = EPW * NW
NP = ((N + 127) // 128) * 128
APT = NP // NS
RPT = 320
NP2 = RPT * NW
SEG = 72

_f32 = jnp.float32


def _make_scatter(D, ch, rb):
    nch = EPW // ch

    def body(h_hbm, src_hbm, dst_hbm, zero_hbm, out_hbm, *refs):
        src_all, dst_all = refs[0], refs[1]
        rows = refs[2:2 + rb]
        acc = refs[2 + rb]
        zsem = refs[3 + rb]
        sems = refs[4 + rb:4 + 2 * rb]
        c = lax.axis_index("c")
        s = lax.axis_index("s")
        wid = c * NS + s
        pltpu.async_copy(zero_hbm.at[pl.ds(s * APT, APT)],
                         acc.at[pl.ds(s * APT, APT)], zsem)
        pltpu.sync_copy(src_hbm.at[wid], src_all)
        pltpu.sync_copy(dst_hbm.at[wid], dst_all)
        pltpu.make_async_copy(zero_hbm.at[pl.ds(s * APT, APT)],
                              acc.at[pl.ds(s * APT, APT)], zsem).wait()
        plsc.subcore_barrier()

        for r in range(rb):
            pltpu.async_copy(h_hbm.at[src_all.at[r]], rows[r], sems[r])

        def do_chunk(k, row_ref, sem):
            @pl.when(k < nch)
            def _():
                pltpu.make_async_copy(h_hbm.at[src_all.at[k]], row_ref,
                                      sem).wait()
                pltpu.sync_copy(row_ref, acc.at[dst_all.at[k]], add=True)

                @pl.when(k + rb < nch)
                def _():
                    pltpu.async_copy(h_hbm.at[src_all.at[k + rb]], row_ref,
                                     sem)

        def step(k2, carry):
            for r in range(rb):
                do_chunk(k2 * rb + r, rows[r], sems[r])
            return carry

        lax.fori_loop(0, (nch + rb - 1) // rb, step, 0)
        plsc.subcore_barrier()
        pltpu.sync_copy(acc.at[pl.ds(s * APT, APT)],
                        out_hbm.at[c, pl.ds(s * APT, APT)])

    return pl.kernel(
        body,
        out_type=jax.ShapeDtypeStruct((NC, NP, D), _f32),
        mesh=plsc.VectorSubcoreMesh(core_axis_name="c", subcore_axis_name="s"),
        scratch_types=(
            [pltpu.VMEM((nch, ch), jnp.int32),
             pltpu.VMEM((nch, ch), jnp.int32)]
            + [pltpu.VMEM((ch, D), _f32) for _ in range(rb)]
            + [pltpu.VMEM_SHARED((NP, D), _f32)]
            + [pltpu.SemaphoreType.DMA for _ in range(rb + 1)]
        ),
        compiler_params=pltpu.CompilerParams(use_tc_tiling_on_sc=False),
    )


CH128 = 64
_sc_scatter128 = _make_scatter(DF, CH128, 3)
_sc_scatter64 = _make_scatter(DE, CH, 4)


def _sc_segmax_body(h0, h1, h2, h3, bat_hbm, o0, o1, o2, o3, batv, buf, acc):
    c = lax.axis_index("c")
    s = lax.axis_index("s")
    wid = c * NS + s
    r0 = wid * RPT
    pltpu.sync_copy(bat_hbm.at[pl.ds(r0, RPT)], batv.at[pl.ds(0, RPT)])
    neg = jnp.full((16,), -jnp.inf, dtype=_f32)
    for h_hbm, o_hbm in ((h0, o0), (h1, o1), (h2, o2), (h3, o3)):
        pltpu.sync_copy(h_hbm.at[pl.ds(r0, RPT)], buf)

        def init(i, carry):
            for q in range(DE // 16):
                acc[i, pl.ds(q * 16, 16)] = neg
            return carry

        lax.fori_loop(0, SEG, init, 0)

        def row(j, carry):
            g = batv[pl.ds(j, 16)][0]
            for q in range(DE // 16):
                acc[g, pl.ds(q * 16, 16)] = jnp.maximum(
                    acc[g, pl.ds(q * 16, 16)], buf[j, pl.ds(q * 16, 16)])
            return carry

        lax.fori_loop(0, RPT, row, 0)
        pltpu.sync_copy(acc, o_hbm.at[wid])


_pool_shape = jax.ShapeDtypeStruct((NW, SEG, DE), _f32)
_sc_segmax = pl.kernel(
    _sc_segmax_body,
    out_type=[_pool_shape, _pool_shape, _pool_shape, _pool_shape],
    mesh=plsc.VectorSubcoreMesh(core_axis_name="c", subcore_axis_name="s"),
    scratch_types=[
        pltpu.VMEM((RPT + 16,), jnp.int32),
        pltpu.VMEM((RPT, DE), _f32),
        pltpu.VMEM((SEG, DE), _f32),
    ],
    compiler_params=pltpu.CompilerParams(use_tc_tiling_on_sc=False),
)


def _dot(a, b):
    return jnp.dot(a.astype(jnp.bfloat16), b.astype(jnp.bfloat16),
                   preferred_element_type=_f32)


def _bn_relu(a, g, b):
    mu = jnp.mean(a, axis=0, keepdims=True)
    d = a - mu
    var = jnp.mean(d * d, axis=0, keepdims=True)
    return jnp.maximum(g * (d * lax.rsqrt(var + BN_EPS)) + b, 0.0)


def _tc_layer_body(h_ref, p_ref, w1_ref, b1_ref, g1_ref, be1_ref, w2_ref,
                   b2_ref, g2_ref, be2_ref, fcw_ref, fcb_ref, h_out_ref):
    a = h_ref[pl.ds(0, N), :] + p_ref[0, :N, :] + p_ref[1, :N, :]
    a = _dot(a, w1_ref[...]) + b1_ref[...]
    a = _bn_relu(a, g1_ref[...], be1_ref[...])
    a = _dot(a, w2_ref[...]) + b2_ref[...]
    a = _bn_relu(a, g2_ref[...], be2_ref[...])
    h = jnp.maximum(_dot(a, fcw_ref[...]) + fcb_ref[...], 0.0)
    h_out_ref[pl.ds(0, N), :] = h
    h_out_ref[pl.ds(N, NP2 - N), :] = jnp.zeros((NP2 - N, DE), _f32)


def _tc_layer(h, p, cp, fcw, fcb):
    return pl.pallas_call(
        _tc_layer_body,
        out_shape=jax.ShapeDtypeStruct((NP2, DE), _f32),
    )(h, p, cp['W1'], cp['b1'].reshape(1, DE), cp['g1'].reshape(1, DE),
      cp['be1'].reshape(1, DE), cp['W2'], cp['b2'].reshape(1, DE),
      cp['g2'].reshape(1, DE), cp['be2'].reshape(1, DE),
      fcw, fcb.reshape(1, DE))


def _tc_head_body(p0_ref, p1_ref, p2_ref, p3_ref, w1_ref, b1_ref,
                  w2_ref, b2_ref, o_ref):
    pools = [jnp.max(r[...][:, :G, :], axis=0)
             for r in (p0_ref, p1_ref, p2_ref, p3_ref)]
    hc = jnp.concatenate(pools, axis=1)
    t = jnp.maximum(_dot(hc, w1_ref[...]) + b1_ref[...], 0.0)
    o_ref[...] = _dot(t, w2_ref[...]) + b2_ref[...]


def _tc_head(pools, params):
    return pl.pallas_call(
        _tc_head_body,
        out_shape=jax.ShapeDtypeStruct((G, DT), _f32),
    )(*pools, params['fc1_W'], params['fc1_b'].reshape(1, DE),
      params['fc2_W'], params['fc2_b'].reshape(1, DT))


def kernel(x, edge_index, batch, params):
    src = edge_index[0].astype(jnp.int32)
    dst = edge_index[1].astype(jnp.int32)
    srcp = jnp.concatenate([src, jnp.zeros((EP - E,), jnp.int32)])
    dstp = jnp.concatenate([dst, jnp.full((EP - E,), N, jnp.int32)])
    srcp64 = srcp.reshape(NW, EPW // CH, CH)
    dstp64 = dstp.reshape(NW, EPW // CH, CH)
    srcp128 = srcp.reshape(NW, EPW // CH128, CH128)
    dstp128 = dstp.reshape(NW, EPW // CH128, CH128)
    batp = jnp.concatenate([batch.astype(jnp.int32),
                            jnp.full((NP2 - N,), G, jnp.int32)])
    zeros128 = jnp.zeros((NP, DF), _f32)
    zeros64 = jnp.zeros((NP, DE), _f32)

    convs = params['convs']
    hs = []
    h = x
    for i in range(NUM_LAYERS):
        if i == 0:
            p = _sc_scatter128(h, srcp128, dstp128, zeros128)
        else:
            p = _sc_scatter64(h, srcp64, dstp64, zeros64)
        h = _tc_layer(h, p, convs[i], params['fc_max_W'], params['fc_max_b'])
        hs.append(h)

    pools = _sc_segmax(hs[0], hs[1], hs[2], hs[3], batp)
    return _tc_head(pools, params)


# speedup vs baseline: 6.0618x; 1.0008x over previous
import jax
import jax.numpy as jnp
from jax import lax
from jax.experimental import pallas as pl
from jax.experimental.pallas import tpu as pltpu
from jax.experimental.pallas import tpu_sc as plsc

N = 10000
E = 320000
DF = 128
DE = 64
G = 64
DT = 10
NUM_LAYERS = 4
BN_EPS = 1e-5

NC = 2
NS = 16
NW = NC * NS

CH = 128
EPW = ((E // NW + CH - 1) // CH) * CH
EP = EPW * NW
NP = ((N + 127) // 128) * 128
APT = NP // NS
RPT = 320
NP2 = RPT * NW
SEG = 72

_f32 = jnp.float32


def _make_scatter(D, ch, rb):
    nch = EPW // ch

    def body(h_hbm, src_hbm, dst_hbm, zero_hbm, out_hbm, *refs):
        src_all, dst_all = refs[0], refs[1]
        rows = refs[2:2 + rb]
        acc = refs[2 + rb]
        zsem = refs[3 + rb]
        sems = refs[4 + rb:4 + 2 * rb]
        c = lax.axis_index("c")
        s = lax.axis_index("s")
        wid = c * NS + s
        pltpu.async_copy(zero_hbm.at[pl.ds(s * APT, APT)],
                         acc.at[pl.ds(s * APT, APT)], zsem)
        pltpu.sync_copy(src_hbm.at[wid], src_all)
        pltpu.sync_copy(dst_hbm.at[wid], dst_all)
        pltpu.make_async_copy(zero_hbm.at[pl.ds(s * APT, APT)],
                              acc.at[pl.ds(s * APT, APT)], zsem).wait()
        plsc.subcore_barrier()

        for r in range(rb):
            pltpu.async_copy(h_hbm.at[src_all.at[r]], rows[r], sems[r])

        def do_chunk(k, row_ref, sem):
            @pl.when(k < nch)
            def _():
                pltpu.make_async_copy(h_hbm.at[src_all.at[k]], row_ref,
                                      sem).wait()
                pltpu.sync_copy(row_ref, acc.at[dst_all.at[k]], add=True)

                @pl.when(k + rb < nch)
                def _():
                    pltpu.async_copy(h_hbm.at[src_all.at[k + rb]], row_ref,
                                     sem)

        def step(k2, carry):
            for r in range(rb):
                do_chunk(k2 * rb + r, rows[r], sems[r])
            return carry

        lax.fori_loop(0, (nch + rb - 1) // rb, step, 0)
        plsc.subcore_barrier()
        pltpu.sync_copy(acc.at[pl.ds(s * APT, APT)],
                        out_hbm.at[c, pl.ds(s * APT, APT)])

    return pl.kernel(
        body,
        out_type=jax.ShapeDtypeStruct((NC, NP, D), _f32),
        mesh=plsc.VectorSubcoreMesh(core_axis_name="c", subcore_axis_name="s"),
        scratch_types=(
            [pltpu.VMEM((nch, ch), jnp.int32),
             pltpu.VMEM((nch, ch), jnp.int32)]
            + [pltpu.VMEM((ch, D), _f32) for _ in range(rb)]
            + [pltpu.VMEM_SHARED((NP, D), _f32)]
            + [pltpu.SemaphoreType.DMA for _ in range(rb + 1)]
        ),
        compiler_params=pltpu.CompilerParams(use_tc_tiling_on_sc=False),
    )


CH128 = 64
_sc_scatter128 = _make_scatter(DF, CH128, 3)
_sc_scatter64 = _make_scatter(DE, CH, 6)
EPAD = EP


def _sc_segmax_body(h0, h1, h2, h3, bat_hbm, o0, o1, o2, o3, batv, buf, acc):
    c = lax.axis_index("c")
    s = lax.axis_index("s")
    wid = c * NS + s
    r0 = wid * RPT
    pltpu.sync_copy(bat_hbm.at[pl.ds(r0, RPT)], batv.at[pl.ds(0, RPT)])
    neg = jnp.full((16,), -jnp.inf, dtype=_f32)
    for h_hbm, o_hbm in ((h0, o0), (h1, o1), (h2, o2), (h3, o3)):
        pltpu.sync_copy(h_hbm.at[pl.ds(r0, RPT)], buf)

        def init(i, carry):
            for q in range(DE // 16):
                acc[i, pl.ds(q * 16, 16)] = neg
            return carry

        lax.fori_loop(0, SEG, init, 0)

        def row(j, carry):
            g = batv[pl.ds(j, 16)][0]
            for q in range(DE // 16):
                acc[g, pl.ds(q * 16, 16)] = jnp.maximum(
                    acc[g, pl.ds(q * 16, 16)], buf[j, pl.ds(q * 16, 16)])
            return carry

        lax.fori_loop(0, RPT, row, 0)
        pltpu.sync_copy(acc, o_hbm.at[wid])


_pool_shape = jax.ShapeDtypeStruct((NW, SEG, DE), _f32)
_sc_segmax = pl.kernel(
    _sc_segmax_body,
    out_type=[_pool_shape, _pool_shape, _pool_shape, _pool_shape],
    mesh=plsc.VectorSubcoreMesh(core_axis_name="c", subcore_axis_name="s"),
    scratch_types=[
        pltpu.VMEM((RPT + 16,), jnp.int32),
        pltpu.VMEM((RPT, DE), _f32),
        pltpu.VMEM((SEG, DE), _f32),
    ],
    compiler_params=pltpu.CompilerParams(use_tc_tiling_on_sc=False),
)


def _dot(a, b):
    return jnp.dot(a.astype(jnp.bfloat16), b.astype(jnp.bfloat16),
                   preferred_element_type=_f32)


def _bn_relu(a, g, b):
    mu = jnp.mean(a, axis=0, keepdims=True)
    d = a - mu
    var = jnp.mean(d * d, axis=0, keepdims=True)
    return jnp.maximum(g * (d * lax.rsqrt(var + BN_EPS)) + b, 0.0)


def _tc_layer_body(h_ref, p_ref, w1_ref, b1_ref, g1_ref, be1_ref, w2_ref,
                   b2_ref, g2_ref, be2_ref, fcw_ref, fcb_ref, h_out_ref):
    a = h_ref[pl.ds(0, N), :] + p_ref[0, :N, :] + p_ref[1, :N, :]
    a = _dot(a, w1_ref[...]) + b1_ref[...]
    a = _bn_relu(a, g1_ref[...], be1_ref[...])
    a = _dot(a, w2_ref[...]) + b2_ref[...]
    a = _bn_relu(a, g2_ref[...], be2_ref[...])
    h = jnp.maximum(_dot(a, fcw_ref[...]) + fcb_ref[...], 0.0)
    h_out_ref[pl.ds(0, N), :] = h
    h_out_ref[pl.ds(N, NP2 - N), :] = jnp.zeros((NP2 - N, DE), _f32)


def _tc_layer(h, p, cp, fcw, fcb):
    return pl.pallas_call(
        _tc_layer_body,
        out_shape=jax.ShapeDtypeStruct((NP2, DE), _f32),
    )(h, p, cp['W1'], cp['b1'].reshape(1, DE), cp['g1'].reshape(1, DE),
      cp['be1'].reshape(1, DE), cp['W2'], cp['b2'].reshape(1, DE),
      cp['g2'].reshape(1, DE), cp['be2'].reshape(1, DE),
      fcw, fcb.reshape(1, DE))


def _tc_head_body(p0_ref, p1_ref, p2_ref, p3_ref, w1_ref, b1_ref,
                  w2_ref, b2_ref, o_ref):
    pools = [jnp.max(r[...][:, :G, :], axis=0)
             for r in (p0_ref, p1_ref, p2_ref, p3_ref)]
    hc = jnp.concatenate(pools, axis=1)
    t = jnp.maximum(_dot(hc, w1_ref[...]) + b1_ref[...], 0.0)
    o_ref[...] = _dot(t, w2_ref[...]) + b2_ref[...]


def _tc_head(pools, params):
    return pl.pallas_call(
        _tc_head_body,
        out_shape=jax.ShapeDtypeStruct((G, DT), _f32),
    )(*pools, params['fc1_W'], params['fc1_b'].reshape(1, DE),
      params['fc2_W'], params['fc2_b'].reshape(1, DT))


def kernel(x, edge_index, batch, params):
    src = edge_index[0].astype(jnp.int32)
    dst = edge_index[1].astype(jnp.int32)
    srcp = jnp.concatenate([src, jnp.zeros((EPAD - E,), jnp.int32)])
    dstp = jnp.concatenate([dst, jnp.full((EPAD - E,), N, jnp.int32)])
    srcp64 = srcp.reshape(NW, EPW // CH, CH)
    dstp64 = dstp.reshape(NW, EPW // CH, CH)
    srcp128 = srcp.reshape(NW, EPW // CH128, CH128)
    dstp128 = dstp.reshape(NW, EPW // CH128, CH128)
    batp = jnp.concatenate([batch.astype(jnp.int32),
                            jnp.full((NP2 - N,), G, jnp.int32)])
    zeros128 = jnp.zeros((NP, DF), _f32)
    zeros64 = jnp.zeros((NP, DE), _f32)

    convs = params['convs']
    hs = []
    h = x
    for i in range(NUM_LAYERS):
        if i == 0:
            p = _sc_scatter128(h, srcp128, dstp128, zeros128)
        else:
            p = _sc_scatter64(h, srcp64, dstp64, zeros64)
        h = _tc_layer(h, p, convs[i], params['fc_max_W'], params['fc_max_b'])
        hs.append(h)

    pools = _sc_segmax(hs[0], hs[1], hs[2], hs[3], batp)
    return _tc_head(pools, params)
